# baseline (speedup 1.0000x reference)
import functools
import math

import jax
import jax.numpy as jnp
from jax.experimental import pallas as pl
from jax.experimental.pallas import tpu as pltpu

_HEADS = 8


def _dot_bt(a, b, bf):
    if bf:
        a, b = a.astype(jnp.bfloat16), b.astype(jnp.bfloat16)
    return jax.lax.dot_general(a, b, (((1,), (1,)), ((), ())),
                               preferred_element_type=jnp.float32)


def _dot(a, b, bf):
    if bf:
        a, b = a.astype(jnp.bfloat16), b.astype(jnp.bfloat16)
    return jax.lax.dot_general(a, b, (((1,), (0,)), ((), ())),
                               preferred_element_type=jnp.float32)


def _lin(x, p, bf=False):
    return _dot_bt(x, p['w'][...], bf) + p['b'][...]


def _ln(x, p):
    m = jnp.mean(x, axis=-1, keepdims=True)
    xc = x - m
    v = jnp.mean(xc * xc, axis=-1, keepdims=True)
    return xc * jax.lax.rsqrt(v + 1e-5) * p['g'][...] + p['b'][...]


def _attention(p, q_in, k_in, v_in):
    nq = q_in.shape[0]
    nk = k_in.shape[0]
    big = nq > 256 or nk > 256
    q = _lin(q_in, p['q'], bf=nq > 256)
    k = _lin(k_in, p['k'], bf=nk > 256)
    v = _lin(v_in, p['v'], bf=nk > 256)
    C = q.shape[1]
    hd = C // _HEADS
    scale = 1.0 / math.sqrt(hd)
    lane = jax.lax.broadcasted_iota(jnp.int32, (1, C), 1)
    out = jnp.zeros((nq, C), jnp.float32)
    for h in range(_HEADS):
        mask = ((lane >= h * hd) & (lane < (h + 1) * hd)).astype(jnp.float32)
        if nq <= nk:
            lhs, rhs = q * mask, k
        else:
            lhs, rhs = q, k * mask
        logits = _dot_bt(lhs, rhs, big) * scale
        mx = jnp.max(logits, axis=-1, keepdims=True)
        e = jnp.exp(logits - mx)
        a = e * (1.0 / jnp.sum(e, axis=-1, keepdims=True))
        if nq <= nk:
            out = out + _dot(a, v, big) * mask
        else:
            out = out + _dot(a, v * mask, big)
    return _lin(out, p['o'], bf=nq > 256)


def _body(treedef, n_param, *refs):
    keys_ref, kpe_ref, point_ref = refs[:3]
    param_refs = refs[3:3 + n_param]
    q_out_ref, k_out_ref = refs[3 + n_param:]
    p = jax.tree_util.tree_unflatten(treedef, list(param_refs))

    keys = keys_ref[0]
    kpe = kpe_ref[0]
    point = point_ref[0]
    queries = point
    for i, bp in enumerate(p['blocks']):
        if i == 0:
            queries = _attention(bp['self_attn'], queries, queries, queries)
        else:
            qq = queries + point
            queries = queries + _attention(bp['self_attn'], qq, qq, queries)
        queries = _ln(queries, bp['norm1'])
        qq = queries + point
        kk = keys + kpe
        queries = queries + _attention(bp['cross_t2i'], qq, kk, keys)
        queries = _ln(queries, bp['norm2'])
        h1 = jnp.maximum(_lin(queries, bp['mlp']['lin1']), 0.0)
        queries = queries + _lin(h1, bp['mlp']['lin2'])
        queries = _ln(queries, bp['norm3'])
        qq = queries + point
        kk = keys + kpe
        keys = keys + _attention(bp['cross_i2t'], kk, qq, queries)
        keys = _ln(keys, bp['norm4'])
    qq = queries + point
    kk = keys + kpe
    queries = queries + _attention(p['final_attn'], qq, kk, keys)
    queries = _ln(queries, p['norm_final'])
    q_out_ref[0] = queries
    k_out_ref[0] = keys


@jax.jit
def kernel(image_embedding, image_pe, point_embedding, params):
    bs, c, h, w = image_embedding.shape
    n = h * w
    npt = point_embedding.shape[1]
    keys0 = image_embedding.reshape(bs, c, n).transpose(0, 2, 1)
    kpe0 = image_pe.reshape(bs, c, n).transpose(0, 2, 1)

    flat, treedef = jax.tree_util.tree_flatten(params)
    flat = [f.reshape(1, -1) if f.ndim == 1 else f for f in flat]

    data_specs = [
        pl.BlockSpec((1, n, c), lambda b: (b, 0, 0)),
        pl.BlockSpec((1, n, c), lambda b: (b, 0, 0)),
        pl.BlockSpec((1, npt, c), lambda b: (b, 0, 0)),
    ]
    w_specs = [
        pl.BlockSpec(f.shape, lambda b, nd=f.ndim: (0,) * nd) for f in flat
    ]
    out_specs = [
        pl.BlockSpec((1, npt, c), lambda b: (b, 0, 0)),
        pl.BlockSpec((1, n, c), lambda b: (b, 0, 0)),
    ]
    out_shape = [
        jax.ShapeDtypeStruct((bs, npt, c), jnp.float32),
        jax.ShapeDtypeStruct((bs, n, c), jnp.float32),
    ]
    body = functools.partial(_body, treedef, len(flat))
    qs, ks = pl.pallas_call(
        body,
        grid=(bs,),
        in_specs=data_specs + w_specs,
        out_specs=out_specs,
        out_shape=out_shape,
        compiler_params=pltpu.CompilerParams(
            dimension_semantics=("arbitrary",),
        ),
    )(keys0, kpe0, point_embedding, *flat)
    return qs, ks


# speedup vs baseline: 2.1187x; 2.1187x over previous
import functools
import math

import jax
import jax.numpy as jnp
from jax.experimental import pallas as pl
from jax.experimental.pallas import tpu as pltpu

_HEADS = 8


def _dot_bt(a, b, bf):
    if bf:
        a, b = a.astype(jnp.bfloat16), b.astype(jnp.bfloat16)
    return jax.lax.dot_general(a, b, (((1,), (1,)), ((), ())),
                               preferred_element_type=jnp.float32)


def _dot(a, b, bf):
    if bf:
        a, b = a.astype(jnp.bfloat16), b.astype(jnp.bfloat16)
    return jax.lax.dot_general(a, b, (((1,), (0,)), ((), ())),
                               preferred_element_type=jnp.float32)


def _dot_tt(a, b, bf):
    if bf:
        a, b = a.astype(jnp.bfloat16), b.astype(jnp.bfloat16)
    return jax.lax.dot_general(a, b, (((0,), (0,)), ((), ())),
                               preferred_element_type=jnp.float32)


def _lin(x, p, bf=False):
    return _dot_bt(x, p['w'][...], bf) + p['b'][...]


def _ln(x, p):
    m = jnp.mean(x, axis=-1, keepdims=True)
    xc = x - m
    v = jnp.mean(xc * xc, axis=-1, keepdims=True)
    return xc * jax.lax.rsqrt(v + 1e-5) * p['g'][...] + p['b'][...]


def _masks(C):
    hd = C // _HEADS
    lane = jax.lax.broadcasted_iota(jnp.int32, (1, C), 1)
    return [((lane >= h * hd) & (lane < (h + 1) * hd)).astype(jnp.float32)
            for h in range(_HEADS)]


def _attn_smallq(p, q_in, k_in, v_in, bf):
    q = _lin(q_in, p['q'])
    k = _lin(k_in, p['k'], bf)
    v = _lin(v_in, p['v'], bf)
    nq, C = q.shape
    hd = C // _HEADS
    scale = 1.0 / math.sqrt(hd)
    masks = _masks(C)
    qs = jnp.concatenate([q * m for m in masks], axis=0)
    logits = _dot_bt(qs, k, bf) * scale
    mx = jnp.max(logits, axis=-1, keepdims=True)
    e = jnp.exp(logits - mx)
    a = e * (1.0 / jnp.sum(e, axis=-1, keepdims=True))
    oc = _dot(a, v, bf)
    out = jnp.zeros((nq, C), jnp.float32)
    for h in range(_HEADS):
        out = out + oc[h * nq:(h + 1) * nq] * masks[h]
    return _lin(out, p['o'])


def _attn_bigq(p, q_in, k_in, v_in, bf):
    q = _lin(q_in, p['q'], bf)
    k = _lin(k_in, p['k'])
    v = _lin(v_in, p['v'])
    nk, C = k.shape
    nq = q.shape[0]
    hd = C // _HEADS
    scale = 1.0 / math.sqrt(hd)
    masks = _masks(C)
    ks = jnp.concatenate([k * m for m in masks], axis=0)
    lt = _dot_bt(ks, q, bf) * scale
    out = jnp.zeros((nq, C), jnp.float32)
    for h in range(_HEADS):
        blk = lt[h * nk:(h + 1) * nk]
        mx = jnp.max(blk, axis=0, keepdims=True)
        e = jnp.exp(blk - mx)
        at = e * (1.0 / jnp.sum(e, axis=0, keepdims=True))
        out = out + _dot_tt(at, v * masks[h], bf)
    return _lin(out, p['o'], bf)


def _body(treedef, n_param, *refs):
    keys_ref, kpe_ref, point_ref = refs[:3]
    param_refs = refs[3:3 + n_param]
    q_out_ref, k_out_ref = refs[3 + n_param:]
    p = jax.tree_util.tree_unflatten(treedef, list(param_refs))

    keys = keys_ref[0]
    kpe = kpe_ref[0]
    point = point_ref[0]
    queries = point
    for i, bp in enumerate(p['blocks']):
        if i == 0:
            queries = _attn_smallq(bp['self_attn'], queries, queries,
                                   queries, bf=False)
        else:
            qq = queries + point
            queries = queries + _attn_smallq(bp['self_attn'], qq, qq,
                                             queries, bf=False)
        queries = _ln(queries, bp['norm1'])
        qq = queries + point
        kk = keys + kpe
        queries = queries + _attn_smallq(bp['cross_t2i'], qq, kk, keys,
                                         bf=True)
        queries = _ln(queries, bp['norm2'])
        h1 = jnp.maximum(_lin(queries, bp['mlp']['lin1']), 0.0)
        queries = queries + _lin(h1, bp['mlp']['lin2'])
        queries = _ln(queries, bp['norm3'])
        qq = queries + point
        keys = keys + _attn_bigq(bp['cross_i2t'], kk, qq, queries, bf=True)
        keys = _ln(keys, bp['norm4'])
    qq = queries + point
    kk = keys + kpe
    queries = queries + _attn_smallq(p['final_attn'], qq, kk, keys, bf=True)
    queries = _ln(queries, p['norm_final'])
    q_out_ref[0] = queries
    k_out_ref[0] = keys


@jax.jit
def kernel(image_embedding, image_pe, point_embedding, params):
    bs, c, h, w = image_embedding.shape
    n = h * w
    npt = point_embedding.shape[1]
    keys0 = image_embedding.reshape(bs, c, n).transpose(0, 2, 1)
    kpe0 = image_pe.reshape(bs, c, n).transpose(0, 2, 1)

    flat, treedef = jax.tree_util.tree_flatten(params)
    flat = [f.reshape(1, -1) if f.ndim == 1 else f for f in flat]

    data_specs = [
        pl.BlockSpec((1, n, c), lambda b: (b, 0, 0)),
        pl.BlockSpec((1, n, c), lambda b: (b, 0, 0)),
        pl.BlockSpec((1, npt, c), lambda b: (b, 0, 0)),
    ]
    w_specs = [
        pl.BlockSpec(f.shape, lambda b, nd=f.ndim: (0,) * nd) for f in flat
    ]
    out_specs = [
        pl.BlockSpec((1, npt, c), lambda b: (b, 0, 0)),
        pl.BlockSpec((1, n, c), lambda b: (b, 0, 0)),
    ]
    out_shape = [
        jax.ShapeDtypeStruct((bs, npt, c), jnp.float32),
        jax.ShapeDtypeStruct((bs, n, c), jnp.float32),
    ]
    body = functools.partial(_body, treedef, len(flat))
    qs, ks = pl.pallas_call(
        body,
        grid=(bs,),
        in_specs=data_specs + w_specs,
        out_specs=out_specs,
        out_shape=out_shape,
        compiler_params=pltpu.CompilerParams(
            dimension_semantics=("arbitrary",),
        ),
    )(keys0, kpe0, point_embedding, *flat)
    return qs, ks


# speedup vs baseline: 2.8650x; 1.3522x over previous
import functools
import math

import jax
import jax.numpy as jnp
from jax.experimental import pallas as pl
from jax.experimental.pallas import tpu as pltpu

_HEADS = 8


def _dot_bt(a, b, bf):
    if bf:
        a, b = a.astype(jnp.bfloat16), b.astype(jnp.bfloat16)
    return jax.lax.dot_general(a, b, (((1,), (1,)), ((), ())),
                               preferred_element_type=jnp.float32)


def _dot(a, b, bf):
    if bf:
        a, b = a.astype(jnp.bfloat16), b.astype(jnp.bfloat16)
    return jax.lax.dot_general(a, b, (((1,), (0,)), ((), ())),
                               preferred_element_type=jnp.float32)


def _dot_tt(a, b, bf):
    if bf:
        a, b = a.astype(jnp.bfloat16), b.astype(jnp.bfloat16)
    return jax.lax.dot_general(a, b, (((0,), (0,)), ((), ())),
                               preferred_element_type=jnp.float32)


def _lin(x, p, bf=False):
    return _dot_bt(x, p['w'][...], bf) + p['b'][...]


def _ln(x, p):
    m = jnp.mean(x, axis=-1, keepdims=True)
    xc = x - m
    v = jnp.mean(xc * xc, axis=-1, keepdims=True)
    return xc * jax.lax.rsqrt(v + 1e-5) * p['g'][...] + p['b'][...]


def _masks(C):
    hd = C // _HEADS
    lane = jax.lax.broadcasted_iota(jnp.int32, (1, C), 1)
    return [((lane >= h * hd) & (lane < (h + 1) * hd)).astype(jnp.float32)
            for h in range(_HEADS)]


def _attn_smallq(p, q_in, k_in, v_in, bf):
    q = _lin(q_in, p['q'])
    k = _lin(k_in, p['k'], bf)
    v = _lin(v_in, p['v'], bf)
    nq, C = q.shape
    hd = C // _HEADS
    scale = 1.0 / math.sqrt(hd)
    masks = _masks(C)
    qs = jnp.concatenate([q * m for m in masks], axis=0)
    logits = _dot_bt(qs, k, bf) * scale
    mx = jnp.max(logits, axis=-1, keepdims=True)
    e = jnp.exp(logits - mx)
    a = e * (1.0 / jnp.sum(e, axis=-1, keepdims=True))
    oc = _dot(a, v, bf)
    out = jnp.zeros((nq, C), jnp.float32)
    for h in range(_HEADS):
        out = out + oc[h * nq:(h + 1) * nq] * masks[h]
    return _lin(out, p['o'])


def _attn_bigq(p, q_in, k_in, v_in, bf):
    q = _lin(q_in, p['q'], bf)
    k = _lin(k_in, p['k'])
    v = _lin(v_in, p['v'])
    nk, C = k.shape
    nq = q.shape[0]
    hd = C // _HEADS
    scale = 1.0 / math.sqrt(hd)
    masks = _masks(C)
    ks = jnp.concatenate([k * m for m in masks], axis=0)
    lt = _dot_bt(ks, q, bf) * scale
    ats = []
    for h in range(_HEADS):
        blk = lt[h * nk:(h + 1) * nk]
        mx = jnp.max(blk, axis=0, keepdims=True)
        e = jnp.exp(blk - mx)
        ats.append(e * (1.0 / jnp.sum(e, axis=0, keepdims=True)))
    at_full = jnp.concatenate(ats, axis=0)
    vs = jnp.concatenate([v * m for m in masks], axis=0)
    out = _dot_tt(at_full, vs, bf)
    return _lin(out, p['o'], bf)


def _body(treedef, n_param, *refs):
    keys_ref, kpe_ref, point_ref = refs[:3]
    param_refs = refs[3:3 + n_param]
    q_out_ref, k_out_ref = refs[3 + n_param:]
    p = jax.tree_util.tree_unflatten(treedef, list(param_refs))

    keys = keys_ref[0]
    kpe = kpe_ref[0]
    point = point_ref[0]
    queries = point
    for i, bp in enumerate(p['blocks']):
        if i == 0:
            queries = _attn_smallq(bp['self_attn'], queries, queries,
                                   queries, bf=False)
        else:
            qq = queries + point
            queries = queries + _attn_smallq(bp['self_attn'], qq, qq,
                                             queries, bf=False)
        queries = _ln(queries, bp['norm1'])
        qq = queries + point
        kk = keys + kpe
        queries = queries + _attn_smallq(bp['cross_t2i'], qq, kk, keys,
                                         bf=True)
        queries = _ln(queries, bp['norm2'])
        h1 = jnp.maximum(_lin(queries, bp['mlp']['lin1']), 0.0)
        queries = queries + _lin(h1, bp['mlp']['lin2'])
        queries = _ln(queries, bp['norm3'])
        qq = queries + point
        keys = keys + _attn_bigq(bp['cross_i2t'], kk, qq, queries, bf=True)
        keys = _ln(keys, bp['norm4'])
    qq = queries + point
    kk = keys + kpe
    queries = queries + _attn_smallq(p['final_attn'], qq, kk, keys, bf=True)
    queries = _ln(queries, p['norm_final'])
    q_out_ref[0] = queries
    k_out_ref[0] = keys


@jax.jit
def kernel(image_embedding, image_pe, point_embedding, params):
    bs, c, h, w = image_embedding.shape
    n = h * w
    npt = point_embedding.shape[1]
    keys0 = image_embedding.reshape(bs, c, n).transpose(0, 2, 1)
    kpe0 = image_pe.reshape(bs, c, n).transpose(0, 2, 1)

    flat, treedef = jax.tree_util.tree_flatten(params)
    flat = [f.reshape(1, -1) if f.ndim == 1 else f for f in flat]

    data_specs = [
        pl.BlockSpec((1, n, c), lambda b: (b, 0, 0)),
        pl.BlockSpec((1, n, c), lambda b: (b, 0, 0)),
        pl.BlockSpec((1, npt, c), lambda b: (b, 0, 0)),
    ]
    w_specs = [
        pl.BlockSpec(f.shape, lambda b, nd=f.ndim: (0,) * nd) for f in flat
    ]
    out_specs = [
        pl.BlockSpec((1, npt, c), lambda b: (b, 0, 0)),
        pl.BlockSpec((1, n, c), lambda b: (b, 0, 0)),
    ]
    out_shape = [
        jax.ShapeDtypeStruct((bs, npt, c), jnp.float32),
        jax.ShapeDtypeStruct((bs, n, c), jnp.float32),
    ]
    body = functools.partial(_body, treedef, len(flat))
    qs, ks = pl.pallas_call(
        body,
        grid=(bs,),
        in_specs=data_specs + w_specs,
        out_specs=out_specs,
        out_shape=out_shape,
        compiler_params=pltpu.CompilerParams(
            dimension_semantics=("arbitrary",),
        ),
    )(keys0, kpe0, point_embedding, *flat)
    return qs, ks
